# baseline (speedup 1.0000x reference)
import functools

import jax
import jax.numpy as jnp
from jax import lax
from jax.experimental import pallas as pl
from jax.experimental.pallas import tpu as pltpu
from jax.experimental.pallas import tpu_sc as plsc

VOCAB = 100000
EMBED_DIM = 64
BATCH = 4096
SEQ = 50
TOTAL = BATCH * SEQ

_NC = 2
_NS = 16
_NW = _NC * _NS

B_PER_W = TOTAL // _NW
SUB = 128
CHUNK = 640
NSUB = CHUNK // SUB
NCHUNK = B_PER_W // CHUNK
NBUF = 2


def _gather_body(idx_hbm, table_hbm, out_hbm, idx_v, rows_v,
                 sem_i, sem_g, sem_s):
    wid = lax.axis_index("s") * _NC + lax.axis_index("c")
    base = wid * B_PER_W

    def idx_load(i):
        return pltpu.async_copy(
            idx_hbm.at[pl.ds(base + i * CHUNK, CHUNK)],
            idx_v.at[i % NBUF], sem_i)

    idx_loads = {i: idx_load(i) for i in range(min(NBUF, NCHUNK))}
    stores = {}
    for i in range(NCHUNK):
        p = i % NBUF
        idx_loads[i].wait()
        if i >= NBUF:
            stores[i - NBUF].wait()
        gathers = [
            pltpu.async_copy(
                table_hbm.at[idx_v.at[p, pl.ds(j * SUB, SUB)]],
                rows_v.at[p, pl.ds(j * SUB, SUB)], sem_g)
            for j in range(NSUB)
        ]
        for g in gathers:
            g.wait()
        if i + NBUF < NCHUNK:
            idx_loads[i + NBUF] = idx_load(i + NBUF)
        stores[i] = pltpu.async_copy(
            rows_v.at[p], out_hbm.at[pl.ds(base + i * CHUNK, CHUNK)], sem_s)
    for i in range(max(0, NCHUNK - NBUF), NCHUNK):
        stores[i].wait()


@jax.jit
def _embedding_lookup(idx_flat, table):
    mesh = plsc.VectorSubcoreMesh(core_axis_name="c", subcore_axis_name="s")
    out = pl.kernel(
        _gather_body,
        out_type=jax.ShapeDtypeStruct((TOTAL, EMBED_DIM), jnp.float32),
        mesh=mesh,
        scratch_types=[
            pltpu.VMEM((NBUF, CHUNK), jnp.int32),
            pltpu.VMEM((NBUF, CHUNK, EMBED_DIM), jnp.float32),
            pltpu.SemaphoreType.DMA,
            pltpu.SemaphoreType.DMA,
            pltpu.SemaphoreType.DMA,
        ],
        compiler_params=pltpu.CompilerParams(use_tc_tiling_on_sc=False),
    )(idx_flat, table)
    return out


def kernel(token_type_ids, table):
    idx_flat = jnp.reshape(token_type_ids, (TOTAL,)).astype(jnp.int32)
    out = _embedding_lookup(idx_flat, table)
    return jnp.reshape(out, (BATCH, SEQ, EMBED_DIM))
